# Initial kernel scaffold; baseline (speedup 1.0000x reference)
import jax
import jax.numpy as jnp
from jax.experimental import pallas as pl


def kernel(indices, weight):
    raise NotImplementedError("write your pallas kernel here")


# speedup vs baseline: 2.3546x; 2.3546x over previous
import jax
import jax.numpy as jnp
from jax.experimental import pallas as pl
from jax.experimental.pallas import tpu as pltpu
from jax.experimental.pallas import tpu_sc as plsc

_SPLIT = 2
_WINDOW = 128


def kernel(indices, weight):
    B, L = indices.shape
    V, D = weight.shape
    Ds = D // _SPLIT
    N = B * L
    NS = N * _SPLIT

    w_split = weight.reshape(V, _SPLIT, Ds).swapaxes(0, 1).reshape(_SPLIT * V, Ds)
    idx2 = (indices.reshape(N, 1) + jnp.arange(_SPLIT, dtype=indices.dtype) * V)
    idx2 = idx2.reshape(1, NS)

    mesh = plsc.VectorSubcoreMesh(core_axis_name="core",
                                  subcore_axis_name="subcore")

    @pl.kernel(out_type=jax.ShapeDtypeStruct((NS, Ds), weight.dtype), mesh=mesh)
    def sc_gather(w_hbm, i_hbm, o_hbm):
        def body(i_vmem, o_vmem):
            pltpu.sync_copy(w_hbm.at[i_vmem.at[0]], o_vmem)

        pltpu.emit_pipeline(
            body,
            grid=(NS // _WINDOW,),
            in_specs=[pl.BlockSpec((1, _WINDOW), index_map=lambda i: (0, i))],
            out_specs=[pl.BlockSpec((_WINDOW, Ds), index_map=lambda i: (i, 0))],
            core_axis_name=("core", "subcore"),
            dimension_semantics=(pltpu.PARALLEL,),
        )(i_hbm, o_hbm)

    out = sc_gather(w_split, idx2)
    return out.reshape(B, L, D)
